# baseline (speedup 1.0000x reference)
import functools

import jax
import jax.numpy as jnp
from jax import lax
from jax.experimental import pallas as pl
from jax.experimental.pallas import tpu as pltpu
from jax.experimental.pallas import tpu_sc as plsc

N_NODES = 10000
N_EDGES = 320000
D = 128

NC = 2
NS = 16
NW = NC * NS

CHUNK = 64
CH_PER_W = 160
NCHUNK = NW * CH_PER_W
E_PAD = NCHUNK * CHUNK
NSLOT = 4
LOOK = 2
N_PHASE = 4
K_PH = CH_PER_W // N_PHASE
N_REAL_CHUNK = N_EDGES // CHUNK
ROWS_PER_SUB = 624
TAIL_ROWS = N_NODES - NS * ROWS_PER_SUB


def _sc_aggregate(x, src2d, dst2d, zeros):
    mesh = plsc.VectorSubcoreMesh(core_axis_name="c", subcore_axis_name="s")

    @functools.partial(
        pl.kernel,
        mesh=mesh,
        out_type=jax.ShapeDtypeStruct((NC, N_NODES, D), jnp.float32),
        scratch_types=[
            pltpu.VMEM((K_PH, CHUNK), jnp.int32),
            pltpu.VMEM((K_PH, CHUNK), jnp.int32),
            pltpu.VMEM((NSLOT, CHUNK, D), jnp.float32),
            pltpu.VMEM_SHARED((N_NODES, D), jnp.float32),
            [pltpu.SemaphoreType.DMA] * NSLOT,
            [pltpu.SemaphoreType.DMA] * NSLOT,
        ],
    )
    def agg(x_hbm, src_hbm, dst_hbm, zeros_hbm, out_hbm,
            src_v, dst_v, rows_v, acc, sem_g, sem_s):
        c = lax.axis_index("c")
        s = lax.axis_index("s")
        wid = s * NC + c

        row0 = s * ROWS_PER_SUB
        pltpu.sync_copy(zeros_hbm.at[pl.ds(row0, ROWS_PER_SUB)],
                        acc.at[pl.ds(row0, ROWS_PER_SUB)])

        @pl.when(s == NS - 1)
        def _():
            t0 = NS * ROWS_PER_SUB
            pltpu.sync_copy(zeros_hbm.at[pl.ds(t0, TAIL_ROWS)],
                            acc.at[pl.ds(t0, TAIL_ROWS)])

        plsc.subcore_barrier()

        c0 = wid * CH_PER_W
        for p in range(N_PHASE):
            pltpu.sync_copy(src_hbm.at[pl.ds(c0 + p * K_PH, K_PH)], src_v)
            pltpu.sync_copy(dst_hbm.at[pl.ds(c0 + p * K_PH, K_PH)], dst_v)

            ph0 = c0 + p * K_PH
            nreal = N_REAL_CHUNK - ph0

            for l in range(LOOK):
                @pl.when(l < nreal)
                def _():
                    pltpu.async_copy(x_hbm.at[src_v.at[l]],
                                     rows_v.at[l], sem_g[l])

            def step(j, b, lookahead):
                pltpu.make_async_copy(x_hbm.at[src_v.at[j]],
                                      rows_v.at[b], sem_g[b]).wait()
                pltpu.async_copy(rows_v.at[b], acc.at[dst_v.at[j]],
                                 sem_s[b], add=True)
                nxt = (b + LOOK) % NSLOT

                @pl.when(j >= NSLOT - LOOK)
                def _():
                    pltpu.make_async_copy(rows_v.at[nxt], acc.at[dst_v.at[j]],
                                          sem_s[nxt]).wait()

                if lookahead:
                    @pl.when(j + LOOK < nreal)
                    def _():
                        pltpu.async_copy(x_hbm.at[src_v.at[j + LOOK]],
                                         rows_v.at[nxt], sem_g[nxt])

            def body(i, carry):
                for b in range(NSLOT):
                    j = NSLOT * i + b

                    @pl.when(j < nreal)
                    def _():
                        step(j, b, lookahead=True)

                return carry

            lax.fori_loop(0, K_PH // NSLOT - 1, body, 0)

            for b in range(NSLOT):
                j = K_PH - NSLOT + b

                @pl.when(j < nreal)
                def _():
                    step(j, b, lookahead=b < NSLOT - LOOK)

            for b in range(LOOK, NSLOT):
                @pl.when(nreal > 0)
                def _():
                    pltpu.make_async_copy(rows_v.at[b], acc.at[dst_v.at[b]],
                                          sem_s[b]).wait()

        plsc.subcore_barrier()

        pltpu.sync_copy(acc.at[pl.ds(row0, ROWS_PER_SUB)],
                        out_hbm.at[c, pl.ds(row0, ROWS_PER_SUB)])

        @pl.when(s == NS - 1)
        def _():
            t0 = NS * ROWS_PER_SUB
            pltpu.sync_copy(acc.at[pl.ds(t0, TAIL_ROWS)],
                            out_hbm.at[c, pl.ds(t0, TAIL_ROWS)])

    return agg(x, src2d, dst2d, zeros)


def _mlp_block(x_ref, p0_ref, p1_ref, w1_ref, b1_ref, w2_ref, b2_ref, o_ref):
    h = x_ref[...] + p0_ref[...] + p1_ref[...]
    h = jnp.dot(h, w1_ref[...], preferred_element_type=jnp.float32) + b1_ref[...]
    h = jnp.maximum(h, 0.0)
    o_ref[...] = (
        jnp.dot(h, w2_ref[...], preferred_element_type=jnp.float32) + b2_ref[...]
    )


def _tc_mlp(x, p0, p1, W1, b1, W2, b2):
    blk = 1000
    grid = (N_NODES // blk,)
    row_spec = pl.BlockSpec((blk, D), lambda i: (i, 0))
    full_spec = pl.BlockSpec((D, D), lambda i: (0, 0))
    bias_spec = pl.BlockSpec((1, D), lambda i: (0, 0))
    return pl.pallas_call(
        _mlp_block,
        grid=grid,
        in_specs=[row_spec, row_spec, row_spec,
                  full_spec, bias_spec, full_spec, bias_spec],
        out_specs=row_spec,
        out_shape=jax.ShapeDtypeStruct((N_NODES, D), jnp.float32),
    )(x, p0, p1, W1.T, b1.reshape(1, D), W2.T, b2.reshape(1, D))


def kernel(x, edge_index, W1, b1, W2, b2):
    src = edge_index[0].astype(jnp.int32)
    dst = edge_index[1].astype(jnp.int32)
    pad = E_PAD - N_EDGES
    src2d = jnp.concatenate([src, jnp.zeros((pad,), jnp.int32)]).reshape(
        NCHUNK, CHUNK)
    dst2d = jnp.concatenate(
        [dst, jnp.zeros((pad,), jnp.int32)]).reshape(NCHUNK, CHUNK)
    zeros = jnp.zeros((N_NODES, D), jnp.float32)
    partials = _sc_aggregate(x, src2d, dst2d, zeros)
    return _tc_mlp(x, partials[0], partials[1], W1, b1, W2, b2)


# speedup vs baseline: 1.0153x; 1.0153x over previous
import functools

import jax
import jax.numpy as jnp
from jax import lax
from jax.experimental import pallas as pl
from jax.experimental.pallas import tpu as pltpu
from jax.experimental.pallas import tpu_sc as plsc

N_NODES = 10000
N_EDGES = 320000
D = 128

NC = 2
NS = 16
NW = NC * NS

CHUNK = 128
CH_PER_W = 80
NCHUNK = NW * CH_PER_W
E_PAD = NCHUNK * CHUNK
NSLOT = 2
LOOK = 1
N_PHASE = 2
K_PH = CH_PER_W // N_PHASE
N_REAL_CHUNK = N_EDGES // CHUNK
ROWS_PER_SUB = 624
TAIL_ROWS = N_NODES - NS * ROWS_PER_SUB


def _sc_aggregate(x, src2d, dst2d):
    mesh = plsc.VectorSubcoreMesh(core_axis_name="c", subcore_axis_name="s")

    @functools.partial(
        pl.kernel,
        mesh=mesh,
        out_type=jax.ShapeDtypeStruct((NC, N_NODES, D), jnp.float32),
        scratch_types=[
            pltpu.VMEM((K_PH, CHUNK), jnp.int32),
            pltpu.VMEM((K_PH, CHUNK), jnp.int32),
            pltpu.VMEM((NSLOT, CHUNK, D), jnp.float32),
            pltpu.VMEM_SHARED((N_NODES, D), jnp.float32),
            [pltpu.SemaphoreType.DMA] * NSLOT,
            [pltpu.SemaphoreType.DMA] * NSLOT,
        ],
    )
    def agg(x_hbm, src_hbm, dst_hbm, out_hbm,
            src_v, dst_v, rows_v, acc, sem_g, sem_s):
        c = lax.axis_index("c")
        s = lax.axis_index("s")
        wid = s * NC + c

        row0 = s * ROWS_PER_SUB
        t0 = NS * ROWS_PER_SUB

        @pl.when(c == 0)
        def _():
            pltpu.sync_copy(x_hbm.at[pl.ds(row0, ROWS_PER_SUB)],
                            acc.at[pl.ds(row0, ROWS_PER_SUB)])

            @pl.when(s == NS - 1)
            def _():
                pltpu.sync_copy(x_hbm.at[pl.ds(t0, TAIL_ROWS)],
                                acc.at[pl.ds(t0, TAIL_ROWS)])

        @pl.when(c == 1)
        def _():
            z16 = jnp.zeros((16,), jnp.float32)

            def zrow(r, carry):
                for col in range(D // 16):
                    rows_v[0, r, pl.ds(col * 16, 16)] = z16
                return carry

            lax.fori_loop(0, CHUNK, zrow, 0)
            for k in range(ROWS_PER_SUB // CHUNK):
                pltpu.sync_copy(rows_v.at[0],
                                acc.at[pl.ds(row0 + k * CHUNK, CHUNK)])
            rem = ROWS_PER_SUB % CHUNK
            pltpu.sync_copy(
                rows_v.at[0, pl.ds(0, rem)],
                acc.at[pl.ds(row0 + ROWS_PER_SUB - rem, rem)])

            @pl.when(s == NS - 1)
            def _():
                pltpu.sync_copy(rows_v.at[0, pl.ds(0, TAIL_ROWS)],
                                acc.at[pl.ds(t0, TAIL_ROWS)])

        plsc.subcore_barrier()

        c0 = wid * CH_PER_W
        for p in range(N_PHASE):
            ph0 = c0 + p * K_PH
            nreal = N_REAL_CHUNK - ph0

            @pl.when(nreal > 0)
            def _():
                pltpu.sync_copy(src_hbm.at[pl.ds(ph0, K_PH)], src_v)
                pltpu.sync_copy(dst_hbm.at[pl.ds(ph0, K_PH)], dst_v)

            for l in range(LOOK):
                @pl.when(l < nreal)
                def _():
                    pltpu.async_copy(x_hbm.at[src_v.at[l]],
                                     rows_v.at[l], sem_g[l])

            def step(j, b, lookahead):
                pltpu.make_async_copy(x_hbm.at[src_v.at[j]],
                                      rows_v.at[b], sem_g[b]).wait()
                pltpu.async_copy(rows_v.at[b], acc.at[dst_v.at[j]],
                                 sem_s[b], add=True)
                nxt = (b + LOOK) % NSLOT

                @pl.when(j >= NSLOT - LOOK)
                def _():
                    pltpu.make_async_copy(rows_v.at[nxt], acc.at[dst_v.at[j]],
                                          sem_s[nxt]).wait()

                if lookahead:
                    @pl.when(j + LOOK < nreal)
                    def _():
                        pltpu.async_copy(x_hbm.at[src_v.at[j + LOOK]],
                                         rows_v.at[nxt], sem_g[nxt])

            def body(i, carry):
                for b in range(NSLOT):
                    j = NSLOT * i + b

                    @pl.when(j < nreal)
                    def _():
                        step(j, b, lookahead=True)

                return carry

            lax.fori_loop(0, K_PH // NSLOT - 1, body, 0)

            for b in range(NSLOT):
                j = K_PH - NSLOT + b

                @pl.when(j < nreal)
                def _():
                    step(j, b, lookahead=b < NSLOT - LOOK)

            for b in range(LOOK, NSLOT):
                @pl.when(nreal > 0)
                def _():
                    pltpu.make_async_copy(rows_v.at[b], acc.at[dst_v.at[b]],
                                          sem_s[b]).wait()

        plsc.subcore_barrier()

        pltpu.sync_copy(acc.at[pl.ds(row0, ROWS_PER_SUB)],
                        out_hbm.at[c, pl.ds(row0, ROWS_PER_SUB)])

        @pl.when(s == NS - 1)
        def _():
            pltpu.sync_copy(acc.at[pl.ds(t0, TAIL_ROWS)],
                            out_hbm.at[c, pl.ds(t0, TAIL_ROWS)])

    return agg(x, src2d, dst2d)


def _mlp_block(p0_ref, p1_ref, w1_ref, b1_ref, w2_ref, b2_ref, o_ref):
    h = p0_ref[...] + p1_ref[...]
    h = jnp.dot(h, w1_ref[...], preferred_element_type=jnp.float32) + b1_ref[...]
    h = jnp.maximum(h, 0.0)
    o_ref[...] = (
        jnp.dot(h, w2_ref[...], preferred_element_type=jnp.float32) + b2_ref[...]
    )


def _tc_mlp(p0, p1, W1, b1, W2, b2):
    blk = 1000
    grid = (N_NODES // blk,)
    row_spec = pl.BlockSpec((blk, D), lambda i: (i, 0))
    full_spec = pl.BlockSpec((D, D), lambda i: (0, 0))
    bias_spec = pl.BlockSpec((1, D), lambda i: (0, 0))
    return pl.pallas_call(
        _mlp_block,
        grid=grid,
        in_specs=[row_spec, row_spec,
                  full_spec, bias_spec, full_spec, bias_spec],
        out_specs=row_spec,
        out_shape=jax.ShapeDtypeStruct((N_NODES, D), jnp.float32),
    )(p0, p1, W1.T, b1.reshape(1, D), W2.T, b2.reshape(1, D))


def kernel(x, edge_index, W1, b1, W2, b2):
    src = edge_index[0].astype(jnp.int32)
    dst = edge_index[1].astype(jnp.int32)
    pad = E_PAD - N_EDGES
    src2d = jnp.concatenate([src, jnp.zeros((pad,), jnp.int32)]).reshape(
        NCHUNK, CHUNK)
    dst2d = jnp.concatenate(
        [dst, jnp.zeros((pad,), jnp.int32)]).reshape(NCHUNK, CHUNK)
    partials = _sc_aggregate(x, src2d, dst2d)
    return _tc_mlp(partials[0], partials[1], W1, b1, W2, b2)


# speedup vs baseline: 1.1483x; 1.1311x over previous
import functools

import jax
import jax.numpy as jnp
from jax import lax
from jax.experimental import pallas as pl
from jax.experimental.pallas import tpu as pltpu
from jax.experimental.pallas import tpu_sc as plsc

N_NODES = 10000
N_EDGES = 320000
D = 128

NC = 2
NS = 16
NW = NC * NS

CHUNK = 128
CH_PER_W = 80
NCHUNK = NW * CH_PER_W
E_PAD = NCHUNK * CHUNK
NSLOT = 2
LOOK = 1
N_PHASE = 2
K_PH = CH_PER_W // N_PHASE
N_REAL_CHUNK = N_EDGES // CHUNK
ROWS_PER_SUB = 624
TAIL_ROWS = N_NODES - NS * ROWS_PER_SUB


def _sc_aggregate(x, src2d, dst2d):
    mesh = plsc.VectorSubcoreMesh(core_axis_name="c", subcore_axis_name="s")

    @functools.partial(
        pl.kernel,
        mesh=mesh,
        out_type=jax.ShapeDtypeStruct((NC, N_NODES, D), jnp.float32),
        scratch_types=[
            pltpu.VMEM((K_PH, CHUNK), jnp.int32),
            pltpu.VMEM((K_PH, CHUNK), jnp.int32),
            pltpu.VMEM((NSLOT, CHUNK, D), jnp.float32),
            pltpu.VMEM_SHARED((N_NODES, D), jnp.float32),
            [pltpu.SemaphoreType.DMA] * NSLOT,
        ],
    )
    def agg(x_hbm, src_hbm, dst_hbm, out_hbm,
            src_v, dst_v, rows_v, acc, sems):
        c = lax.axis_index("c")
        s = lax.axis_index("s")
        wid = s * NC + c

        row0 = s * ROWS_PER_SUB
        t0 = NS * ROWS_PER_SUB

        @pl.when(c == 0)
        def _():
            pltpu.sync_copy(x_hbm.at[pl.ds(row0, ROWS_PER_SUB)],
                            acc.at[pl.ds(row0, ROWS_PER_SUB)])

            @pl.when(s == NS - 1)
            def _():
                pltpu.sync_copy(x_hbm.at[pl.ds(t0, TAIL_ROWS)],
                                acc.at[pl.ds(t0, TAIL_ROWS)])

        @pl.when(c == 1)
        def _():
            z16 = jnp.zeros((16,), jnp.float32)

            def zrow(r, carry):
                for col in range(D // 16):
                    rows_v[0, r, pl.ds(col * 16, 16)] = z16
                return carry

            lax.fori_loop(0, CHUNK, zrow, 0)
            for k in range(ROWS_PER_SUB // CHUNK):
                pltpu.sync_copy(rows_v.at[0],
                                acc.at[pl.ds(row0 + k * CHUNK, CHUNK)])
            rem = ROWS_PER_SUB % CHUNK
            pltpu.sync_copy(
                rows_v.at[0, pl.ds(0, rem)],
                acc.at[pl.ds(row0 + ROWS_PER_SUB - rem, rem)])

            @pl.when(s == NS - 1)
            def _():
                pltpu.sync_copy(rows_v.at[0, pl.ds(0, TAIL_ROWS)],
                                acc.at[pl.ds(t0, TAIL_ROWS)])

        plsc.subcore_barrier()

        c0 = wid * CH_PER_W
        for p in range(N_PHASE):
            ph0 = c0 + p * K_PH
            nreal = N_REAL_CHUNK - ph0

            @pl.when(nreal > 0)
            def _():
                pltpu.sync_copy(src_hbm.at[pl.ds(ph0, K_PH)], src_v)
                pltpu.sync_copy(dst_hbm.at[pl.ds(ph0, K_PH)], dst_v)

            for b in range(NSLOT):
                @pl.when(b < nreal)
                def _():
                    pltpu.async_copy(x_hbm.at[src_v.at[b]],
                                     rows_v.at[b], sems[b])

            def step(j, b, lookahead):
                pltpu.make_async_copy(x_hbm.at[src_v.at[j]],
                                      rows_v.at[b], sems[b]).wait()
                pltpu.async_copy(rows_v.at[b], acc.at[dst_v.at[j]],
                                 sems[b], add=True).wait()

                if lookahead:
                    @pl.when(j + NSLOT < nreal)
                    def _():
                        pltpu.async_copy(x_hbm.at[src_v.at[j + NSLOT]],
                                         rows_v.at[b], sems[b])

            def body(i, carry):
                for b in range(NSLOT):
                    j = NSLOT * i + b

                    @pl.when(j < nreal)
                    def _():
                        step(j, b, lookahead=True)

                return carry

            lax.fori_loop(0, K_PH // NSLOT - 1, body, 0)

            for b in range(NSLOT):
                j = K_PH - NSLOT + b

                @pl.when(j < nreal)
                def _():
                    step(j, b, lookahead=False)

        plsc.subcore_barrier()

        pltpu.sync_copy(acc.at[pl.ds(row0, ROWS_PER_SUB)],
                        out_hbm.at[c, pl.ds(row0, ROWS_PER_SUB)])

        @pl.when(s == NS - 1)
        def _():
            pltpu.sync_copy(acc.at[pl.ds(t0, TAIL_ROWS)],
                            out_hbm.at[c, pl.ds(t0, TAIL_ROWS)])

    return agg(x, src2d, dst2d)


def _mlp_block(p0_ref, p1_ref, w1_ref, b1_ref, w2_ref, b2_ref, o_ref):
    h = p0_ref[...] + p1_ref[...]
    h = jnp.dot(h, w1_ref[...], preferred_element_type=jnp.float32) + b1_ref[...]
    h = jnp.maximum(h, 0.0)
    o_ref[...] = (
        jnp.dot(h, w2_ref[...], preferred_element_type=jnp.float32) + b2_ref[...]
    )


def _tc_mlp(p0, p1, W1, b1, W2, b2):
    blk = 1000
    grid = (N_NODES // blk,)
    row_spec = pl.BlockSpec((blk, D), lambda i: (i, 0))
    full_spec = pl.BlockSpec((D, D), lambda i: (0, 0))
    bias_spec = pl.BlockSpec((1, D), lambda i: (0, 0))
    return pl.pallas_call(
        _mlp_block,
        grid=grid,
        in_specs=[row_spec, row_spec,
                  full_spec, bias_spec, full_spec, bias_spec],
        out_specs=row_spec,
        out_shape=jax.ShapeDtypeStruct((N_NODES, D), jnp.float32),
    )(p0, p1, W1.T, b1.reshape(1, D), W2.T, b2.reshape(1, D))


def kernel(x, edge_index, W1, b1, W2, b2):
    src = edge_index[0].astype(jnp.int32)
    dst = edge_index[1].astype(jnp.int32)
    pad = E_PAD - N_EDGES
    src2d = jnp.concatenate([src, jnp.zeros((pad,), jnp.int32)]).reshape(
        NCHUNK, CHUNK)
    dst2d = jnp.concatenate(
        [dst, jnp.zeros((pad,), jnp.int32)]).reshape(NCHUNK, CHUNK)
    partials = _sc_aggregate(x, src2d, dst2d)
    return _tc_mlp(partials[0], partials[1], W1, b1, W2, b2)


# speedup vs baseline: 1.2266x; 1.0682x over previous
import functools

import jax
import jax.numpy as jnp
from jax import lax
from jax.experimental import pallas as pl
from jax.experimental.pallas import tpu as pltpu
from jax.experimental.pallas import tpu_sc as plsc

N_NODES = 10000
N_EDGES = 320000
D = 128

NC = 2
NS = 16
NW = NC * NS

CHUNK = 128
CH_PER_W = 80
NCHUNK = NW * CH_PER_W
E_PAD = NCHUNK * CHUNK
NSLOT = 2
LOOK = 1
N_PHASE = 2
K_PH = CH_PER_W // N_PHASE
N_REAL_CHUNK = N_EDGES // CHUNK
ROWS_PER_SUB = 624
TAIL_ROWS = N_NODES - NS * ROWS_PER_SUB


def _sc_aggregate(x, src2d, dst2d):
    mesh = plsc.VectorSubcoreMesh(core_axis_name="c", subcore_axis_name="s")

    @functools.partial(
        pl.kernel,
        mesh=mesh,
        out_type=(jax.ShapeDtypeStruct((N_NODES, D), jnp.float32),
                  jax.ShapeDtypeStruct((N_NODES, D), jnp.float32)),
        scratch_types=[
            pltpu.VMEM((K_PH, CHUNK), jnp.int32),
            pltpu.VMEM((K_PH, CHUNK), jnp.int32),
            pltpu.VMEM((NSLOT, CHUNK, D), jnp.float32),
            pltpu.VMEM_SHARED((N_NODES, D), jnp.float32),
            [pltpu.SemaphoreType.DMA] * NSLOT,
        ],
    )
    def agg(x_hbm, src_hbm, dst_hbm, out0_hbm, out1_hbm,
            src_v, dst_v, rows_v, acc, sems):
        c = lax.axis_index("c")
        s = lax.axis_index("s")
        wid = s * NC + c

        row0 = s * ROWS_PER_SUB
        t0 = NS * ROWS_PER_SUB

        @pl.when(c == 0)
        def _():
            pltpu.sync_copy(x_hbm.at[pl.ds(row0, ROWS_PER_SUB)],
                            acc.at[pl.ds(row0, ROWS_PER_SUB)])

            @pl.when(s == NS - 1)
            def _():
                pltpu.sync_copy(x_hbm.at[pl.ds(t0, TAIL_ROWS)],
                                acc.at[pl.ds(t0, TAIL_ROWS)])

        @pl.when(c == 1)
        def _():
            z16 = jnp.zeros((16,), jnp.float32)

            def zrow(r, carry):
                for col in range(D // 16):
                    rows_v[0, r, pl.ds(col * 16, 16)] = z16
                return carry

            lax.fori_loop(0, CHUNK, zrow, 0)
            for k in range(ROWS_PER_SUB // CHUNK):
                pltpu.sync_copy(rows_v.at[0],
                                acc.at[pl.ds(row0 + k * CHUNK, CHUNK)])
            rem = ROWS_PER_SUB % CHUNK
            pltpu.sync_copy(
                rows_v.at[0, pl.ds(0, rem)],
                acc.at[pl.ds(row0 + ROWS_PER_SUB - rem, rem)])

            @pl.when(s == NS - 1)
            def _():
                pltpu.sync_copy(rows_v.at[0, pl.ds(0, TAIL_ROWS)],
                                acc.at[pl.ds(t0, TAIL_ROWS)])

        plsc.subcore_barrier()

        c0 = wid * CH_PER_W
        for p in range(N_PHASE):
            ph0 = c0 + p * K_PH
            nreal = N_REAL_CHUNK - ph0

            @pl.when(nreal > 0)
            def _():
                pltpu.sync_copy(src_hbm.at[pl.ds(ph0, K_PH)], src_v)
                pltpu.sync_copy(dst_hbm.at[pl.ds(ph0, K_PH)], dst_v)

            for b in range(NSLOT):
                @pl.when(b < nreal)
                def _():
                    pltpu.async_copy(x_hbm.at[src_v.at[b]],
                                     rows_v.at[b], sems[b])

            def step(j, b, lookahead):
                pltpu.make_async_copy(x_hbm.at[src_v.at[j]],
                                      rows_v.at[b], sems[b]).wait()
                pltpu.async_copy(rows_v.at[b], acc.at[dst_v.at[j]],
                                 sems[b], add=True).wait()

                if lookahead:
                    @pl.when(j + NSLOT < nreal)
                    def _():
                        pltpu.async_copy(x_hbm.at[src_v.at[j + NSLOT]],
                                         rows_v.at[b], sems[b])

            def body(i, carry):
                for b in range(NSLOT):
                    j = NSLOT * i + b

                    @pl.when(j < nreal)
                    def _():
                        step(j, b, lookahead=True)

                return carry

            lax.fori_loop(0, K_PH // NSLOT - 1, body, 0)

            for b in range(NSLOT):
                j = K_PH - NSLOT + b

                @pl.when(j < nreal)
                def _():
                    step(j, b, lookahead=False)

        plsc.subcore_barrier()

        for ci, out_hbm in ((0, out0_hbm), (1, out1_hbm)):
            @pl.when(c == ci)
            def _():
                pltpu.sync_copy(acc.at[pl.ds(row0, ROWS_PER_SUB)],
                                out_hbm.at[pl.ds(row0, ROWS_PER_SUB)])

                @pl.when(s == NS - 1)
                def _():
                    pltpu.sync_copy(acc.at[pl.ds(t0, TAIL_ROWS)],
                                    out_hbm.at[pl.ds(t0, TAIL_ROWS)])

    return agg(x, src2d, dst2d)


def _mlp_block(p0_ref, p1_ref, w1_ref, b1_ref, w2_ref, b2_ref, o_ref):
    h = p0_ref[...] + p1_ref[...]
    h = jnp.dot(h, w1_ref[...], preferred_element_type=jnp.float32) + b1_ref[...]
    h = jnp.maximum(h, 0.0)
    o_ref[...] = (
        jnp.dot(h, w2_ref[...], preferred_element_type=jnp.float32) + b2_ref[...]
    )


def _tc_mlp(p0, p1, W1, b1, W2, b2):
    blk = 2000
    grid = (N_NODES // blk,)
    row_spec = pl.BlockSpec((blk, D), lambda i: (i, 0))
    full_spec = pl.BlockSpec((D, D), lambda i: (0, 0))
    bias_spec = pl.BlockSpec((1, D), lambda i: (0, 0))
    return pl.pallas_call(
        _mlp_block,
        grid=grid,
        in_specs=[row_spec, row_spec,
                  full_spec, bias_spec, full_spec, bias_spec],
        out_specs=row_spec,
        out_shape=jax.ShapeDtypeStruct((N_NODES, D), jnp.float32),
    )(p0, p1, W1.T, b1.reshape(1, D), W2.T, b2.reshape(1, D))


def kernel(x, edge_index, W1, b1, W2, b2):
    src = edge_index[0].astype(jnp.int32)
    dst = edge_index[1].astype(jnp.int32)
    pad = E_PAD - N_EDGES
    src2d = jnp.concatenate([src, jnp.zeros((pad,), jnp.int32)]).reshape(
        NCHUNK, CHUNK)
    dst2d = jnp.concatenate(
        [dst, jnp.zeros((pad,), jnp.int32)]).reshape(NCHUNK, CHUNK)
    p0, p1 = _sc_aggregate(x, src2d, dst2d)
    return _tc_mlp(p0, p1, W1, b1, W2, b2)


# speedup vs baseline: 1.2448x; 1.0148x over previous
import functools

import jax
import jax.numpy as jnp
from jax import lax
from jax.experimental import pallas as pl
from jax.experimental.pallas import tpu as pltpu
from jax.experimental.pallas import tpu_sc as plsc

N_NODES = 10000
N_EDGES = 320000
D = 128

NC = 2
NS = 16
NW = NC * NS

CHUNK = 80
CH_PER_W = 128
NCHUNK = NW * CH_PER_W
E_PAD = NCHUNK * CHUNK
NSLOT = 4
N_PHASE = 4
K_PH = CH_PER_W // N_PHASE
N_REAL_CHUNK = N_EDGES // CHUNK
ROWS_PER_SUB = 624
TAIL_ROWS = N_NODES - NS * ROWS_PER_SUB


def _sc_aggregate(x, src2d, dst2d):
    mesh = plsc.VectorSubcoreMesh(core_axis_name="c", subcore_axis_name="s")

    @functools.partial(
        pl.kernel,
        mesh=mesh,
        out_type=(jax.ShapeDtypeStruct((N_NODES, D), jnp.float32),
                  jax.ShapeDtypeStruct((N_NODES, D), jnp.float32)),
        scratch_types=[
            pltpu.VMEM((K_PH, CHUNK), jnp.int32),
            pltpu.VMEM((K_PH, CHUNK), jnp.int32),
            pltpu.VMEM((NSLOT, CHUNK, D), jnp.float32),
            pltpu.VMEM_SHARED((N_NODES, D), jnp.float32),
            [pltpu.SemaphoreType.DMA] * NSLOT,
        ],
    )
    def agg(x_hbm, src_hbm, dst_hbm, out0_hbm, out1_hbm,
            src_v, dst_v, rows_v, acc, sems):
        c = lax.axis_index("c")
        s = lax.axis_index("s")
        wid = s * NC + c

        row0 = s * ROWS_PER_SUB
        t0 = NS * ROWS_PER_SUB

        @pl.when(c == 0)
        def _():
            pltpu.sync_copy(x_hbm.at[pl.ds(row0, ROWS_PER_SUB)],
                            acc.at[pl.ds(row0, ROWS_PER_SUB)])

            @pl.when(s == NS - 1)
            def _():
                pltpu.sync_copy(x_hbm.at[pl.ds(t0, TAIL_ROWS)],
                                acc.at[pl.ds(t0, TAIL_ROWS)])

        @pl.when(c == 1)
        def _():
            z16 = jnp.zeros((16,), jnp.float32)

            def zrow(r, carry):
                for col in range(D // 16):
                    rows_v[0, r, pl.ds(col * 16, 16)] = z16
                return carry

            lax.fori_loop(0, CHUNK, zrow, 0)
            for k in range(ROWS_PER_SUB // CHUNK):
                pltpu.sync_copy(rows_v.at[0],
                                acc.at[pl.ds(row0 + k * CHUNK, CHUNK)])
            rem = ROWS_PER_SUB % CHUNK
            pltpu.sync_copy(
                rows_v.at[0, pl.ds(0, rem)],
                acc.at[pl.ds(row0 + ROWS_PER_SUB - rem, rem)])

            @pl.when(s == NS - 1)
            def _():
                pltpu.sync_copy(rows_v.at[0, pl.ds(0, TAIL_ROWS)],
                                acc.at[pl.ds(t0, TAIL_ROWS)])

        plsc.subcore_barrier()

        c0 = wid * CH_PER_W
        for p in range(N_PHASE):
            ph0 = c0 + p * K_PH
            nreal = N_REAL_CHUNK - ph0

            @pl.when(nreal > 0)
            def _():
                pltpu.sync_copy(src_hbm.at[pl.ds(ph0, K_PH)], src_v)
                pltpu.sync_copy(dst_hbm.at[pl.ds(ph0, K_PH)], dst_v)

            for b in range(NSLOT):
                @pl.when(b < nreal)
                def _():
                    pltpu.async_copy(x_hbm.at[src_v.at[b]],
                                     rows_v.at[b], sems[b])

            def step(j, b, lookahead):
                pltpu.make_async_copy(x_hbm.at[src_v.at[j]],
                                      rows_v.at[b], sems[b]).wait()
                pltpu.async_copy(rows_v.at[b], acc.at[dst_v.at[j]],
                                 sems[b], add=True).wait()

                if lookahead:
                    @pl.when(j + NSLOT < nreal)
                    def _():
                        pltpu.async_copy(x_hbm.at[src_v.at[j + NSLOT]],
                                         rows_v.at[b], sems[b])

            def body(i, carry):
                for b in range(NSLOT):
                    j = NSLOT * i + b

                    @pl.when(j < nreal)
                    def _():
                        step(j, b, lookahead=True)

                return carry

            lax.fori_loop(0, K_PH // NSLOT - 1, body, 0)

            for b in range(NSLOT):
                j = K_PH - NSLOT + b

                @pl.when(j < nreal)
                def _():
                    step(j, b, lookahead=False)

        plsc.subcore_barrier()

        for ci, out_hbm in ((0, out0_hbm), (1, out1_hbm)):
            @pl.when(c == ci)
            def _():
                pltpu.sync_copy(acc.at[pl.ds(row0, ROWS_PER_SUB)],
                                out_hbm.at[pl.ds(row0, ROWS_PER_SUB)])

                @pl.when(s == NS - 1)
                def _():
                    pltpu.sync_copy(acc.at[pl.ds(t0, TAIL_ROWS)],
                                    out_hbm.at[pl.ds(t0, TAIL_ROWS)])

    return agg(x, src2d, dst2d)


def _mlp_block(p0_ref, p1_ref, w1_ref, b1_ref, w2_ref, b2_ref, o_ref):
    h = p0_ref[...] + p1_ref[...]
    h = jnp.dot(h, w1_ref[...], preferred_element_type=jnp.float32) + b1_ref[...]
    h = jnp.maximum(h, 0.0)
    o_ref[...] = (
        jnp.dot(h, w2_ref[...], preferred_element_type=jnp.float32) + b2_ref[...]
    )


def _tc_mlp(p0, p1, W1, b1, W2, b2):
    blk = 2000
    grid = (N_NODES // blk,)
    row_spec = pl.BlockSpec((blk, D), lambda i: (i, 0))
    full_spec = pl.BlockSpec((D, D), lambda i: (0, 0))
    bias_spec = pl.BlockSpec((1, D), lambda i: (0, 0))
    return pl.pallas_call(
        _mlp_block,
        grid=grid,
        in_specs=[row_spec, row_spec,
                  full_spec, bias_spec, full_spec, bias_spec],
        out_specs=row_spec,
        out_shape=jax.ShapeDtypeStruct((N_NODES, D), jnp.float32),
    )(p0, p1, W1.T, b1.reshape(1, D), W2.T, b2.reshape(1, D))


def kernel(x, edge_index, W1, b1, W2, b2):
    src = edge_index[0].astype(jnp.int32)
    dst = edge_index[1].astype(jnp.int32)
    pad = E_PAD - N_EDGES
    src2d = jnp.concatenate([src, jnp.zeros((pad,), jnp.int32)]).reshape(
        NCHUNK, CHUNK)
    dst2d = jnp.concatenate(
        [dst, jnp.zeros((pad,), jnp.int32)]).reshape(NCHUNK, CHUNK)
    p0, p1 = _sc_aggregate(x, src2d, dst2d)
    return _tc_mlp(p0, p1, W1, b1, W2, b2)


# speedup vs baseline: 1.2493x; 1.0036x over previous
import functools

import jax
import jax.numpy as jnp
from jax import lax
from jax.experimental import pallas as pl
from jax.experimental.pallas import tpu as pltpu
from jax.experimental.pallas import tpu_sc as plsc

N_NODES = 10000
N_EDGES = 320000
D = 128

NC = 2
NS = 16
NW = NC * NS

CHUNK = 80
CH_PER_W = 128
NCHUNK = NW * CH_PER_W
E_PAD = NCHUNK * CHUNK
NSLOT = 4
N_PHASE = 4
K_PH = CH_PER_W // N_PHASE
N_REAL_CHUNK = N_EDGES // CHUNK
ROWS_PER_SUB = 624
TAIL_ROWS = N_NODES - NS * ROWS_PER_SUB


def _sc_aggregate(x, src2d, dst2d):
    mesh = plsc.VectorSubcoreMesh(core_axis_name="c", subcore_axis_name="s")

    @functools.partial(
        pl.kernel,
        mesh=mesh,
        out_type=(jax.ShapeDtypeStruct((N_NODES, D), jnp.float32),
                  jax.ShapeDtypeStruct((N_NODES, D), jnp.float32)),
        scratch_types=[
            pltpu.VMEM((K_PH, CHUNK), jnp.int32),
            pltpu.VMEM((K_PH, CHUNK), jnp.int32),
            pltpu.VMEM((NSLOT, CHUNK, D), jnp.float32),
            pltpu.VMEM_SHARED((N_NODES, D), jnp.float32),
            [pltpu.SemaphoreType.DMA] * NSLOT,
        ],
    )
    def agg(x_hbm, src_hbm, dst_hbm, out0_hbm, out1_hbm,
            src_v, dst_v, rows_v, acc, sems):
        c = lax.axis_index("c")
        s = lax.axis_index("s")
        wid = s * NC + c

        row0 = s * ROWS_PER_SUB
        t0 = NS * ROWS_PER_SUB

        @pl.when(c == 0)
        def _():
            pltpu.sync_copy(x_hbm.at[pl.ds(row0, ROWS_PER_SUB)],
                            acc.at[pl.ds(row0, ROWS_PER_SUB)])

            @pl.when(s == NS - 1)
            def _():
                pltpu.sync_copy(x_hbm.at[pl.ds(t0, TAIL_ROWS)],
                                acc.at[pl.ds(t0, TAIL_ROWS)])

        @pl.when(c == 1)
        def _():
            z16 = jnp.zeros((16,), jnp.float32)

            def zrow(r, carry):
                for col in range(D // 16):
                    rows_v[0, r, pl.ds(col * 16, 16)] = z16
                return carry

            lax.fori_loop(0, CHUNK, zrow, 0)
            for k in range(ROWS_PER_SUB // CHUNK):
                pltpu.sync_copy(rows_v.at[0],
                                acc.at[pl.ds(row0 + k * CHUNK, CHUNK)])
            rem = ROWS_PER_SUB % CHUNK
            pltpu.sync_copy(
                rows_v.at[0, pl.ds(0, rem)],
                acc.at[pl.ds(row0 + ROWS_PER_SUB - rem, rem)])

            @pl.when(s == NS - 1)
            def _():
                pltpu.sync_copy(rows_v.at[0, pl.ds(0, TAIL_ROWS)],
                                acc.at[pl.ds(t0, TAIL_ROWS)])

        plsc.subcore_barrier()

        c0 = wid * CH_PER_W
        for p in range(N_PHASE):
            ph0 = c0 + p * K_PH
            nreal = N_REAL_CHUNK - ph0

            @pl.when(nreal > 0)
            def _():
                pltpu.sync_copy(src_hbm.at[pl.ds(ph0, K_PH)], src_v)
                pltpu.sync_copy(dst_hbm.at[pl.ds(ph0, K_PH)], dst_v)

            for b in range(NSLOT):
                @pl.when(b < nreal)
                def _():
                    pltpu.async_copy(x_hbm.at[src_v.at[b]],
                                     rows_v.at[b], sems[b])

            def step(j, b, lookahead):
                pltpu.make_async_copy(x_hbm.at[src_v.at[j]],
                                      rows_v.at[b], sems[b]).wait()
                pltpu.async_copy(rows_v.at[b], acc.at[dst_v.at[j]],
                                 sems[b], add=True).wait()

                if lookahead:
                    @pl.when(j + NSLOT < nreal)
                    def _():
                        pltpu.async_copy(x_hbm.at[src_v.at[j + NSLOT]],
                                         rows_v.at[b], sems[b])

            def body(i, carry):
                for b in range(NSLOT):
                    j = NSLOT * i + b

                    @pl.when(j < nreal)
                    def _():
                        step(j, b, lookahead=True)

                return carry

            lax.fori_loop(0, K_PH // NSLOT - 1, body, 0)

            for b in range(NSLOT):
                j = K_PH - NSLOT + b

                @pl.when(j < nreal)
                def _():
                    step(j, b, lookahead=False)

        plsc.subcore_barrier()

        for ci, out_hbm in ((0, out0_hbm), (1, out1_hbm)):
            @pl.when(c == ci)
            def _():
                pltpu.sync_copy(acc.at[pl.ds(row0, ROWS_PER_SUB)],
                                out_hbm.at[pl.ds(row0, ROWS_PER_SUB)])

                @pl.when(s == NS - 1)
                def _():
                    pltpu.sync_copy(acc.at[pl.ds(t0, TAIL_ROWS)],
                                    out_hbm.at[pl.ds(t0, TAIL_ROWS)])

    return agg(x, src2d, dst2d)


def _mlp_block(p0_ref, p1_ref, w1_ref, b1_ref, w2_ref, b2_ref, o_ref):
    h = p0_ref[...] + p1_ref[...]
    h = jnp.dot(h, w1_ref[...], preferred_element_type=jnp.float32) + b1_ref[...]
    h = jnp.maximum(h, 0.0)
    o_ref[...] = (
        jnp.dot(h, w2_ref[...], preferred_element_type=jnp.float32) + b2_ref[...]
    )


def _tc_mlp(p0, p1, W1, b1, W2, b2):
    blk = 2000
    grid = (N_NODES // blk,)
    row_spec = pl.BlockSpec((blk, D), lambda i: (i, 0))
    full_spec = pl.BlockSpec((D, D), lambda i: (0, 0))
    bias_spec = pl.BlockSpec((1, D), lambda i: (0, 0))
    return pl.pallas_call(
        _mlp_block,
        grid=grid,
        in_specs=[row_spec, row_spec,
                  full_spec, bias_spec, full_spec, bias_spec],
        out_specs=row_spec,
        out_shape=jax.ShapeDtypeStruct((N_NODES, D), jnp.float32),
    )(p0, p1, W1.T, b1.reshape(1, D), W2.T, b2.reshape(1, D))


def kernel(x, edge_index, W1, b1, W2, b2):
    src2d = edge_index[0].astype(jnp.int32).reshape(N_REAL_CHUNK, CHUNK)
    dst2d = edge_index[1].astype(jnp.int32).reshape(N_REAL_CHUNK, CHUNK)
    p0, p1 = _sc_aggregate(x, src2d, dst2d)
    return _tc_mlp(p0, p1, W1, b1, W2, b2)
